# Initial kernel scaffold; baseline (speedup 1.0000x reference)
import jax
import jax.numpy as jnp
from jax.experimental import pallas as pl


def kernel(top_indices, distance_emb):
    raise NotImplementedError("write your pallas kernel here")


# speedup vs baseline: 2.2574x; 2.2574x over previous
import functools
import numpy as np
import jax
import jax.numpy as jnp
from jax import lax
from jax.experimental import pallas as pl
from jax.experimental.pallas import tpu as pltpu
from jax.experimental.pallas import tpu_sc as plsc

_NWORDS = 8192
_K = 50
_EMB = 64
_NC, _NS = 2, 16
_NWK = _NC * _NS
_B = _NWORDS * _K
_BPW = _B // _NWK
_PPW = _BPW // 2
_CH = 128
_NCH = _PPW // _CH
_WPW = _BPW // _K

_WE = (2 * np.arange(_PPW, dtype=np.int32)) // _K
_WO = (2 * np.arange(_PPW, dtype=np.int32) + 1) // _K

_mesh = plsc.VectorSubcoreMesh(
    core_axis_name="c", subcore_axis_name="s", num_cores=_NC, num_subcores=_NS
)


def _bucket(word, top):
    d = jnp.maximum(word - top, 1)
    one = jnp.int32(1)
    zero = jnp.int32(0)
    bl = (
        4
        + jnp.where(d >= 8, one, zero)
        + jnp.where(d >= 16, one, zero)
        + jnp.where(d >= 32, one, zero)
        + jnp.where(d >= 64, one, zero)
    )
    return jnp.where(d < 5, d - 1, bl)


def _body(tope_hbm, topo_hbm, we_hbm, wo_hbm, ptab_hbm, out_hbm,
          tope_v, topo_v, we_v, wo_v, idx_v, rows_v, sem):
    wid = lax.axis_index("s") * _NC + lax.axis_index("c")
    pbase = wid * _PPW
    pltpu.sync_copy(tope_hbm.at[pl.ds(pbase, _PPW)], tope_v)
    pltpu.sync_copy(topo_hbm.at[pl.ds(pbase, _PPW)], topo_v)
    pltpu.sync_copy(we_hbm, we_v)
    pltpu.sync_copy(wo_hbm, wo_v)
    wbase = wid * _WPW

    def chunk(c, carry):
        p0 = c * _CH
        for g in range(_CH // 16):
            j = p0 + g * 16
            te = tope_v[pl.ds(j, 16)]
            to = topo_v[pl.ds(j, 16)]
            we = we_v[pl.ds(j, 16)] + wbase
            wo = wo_v[pl.ds(j, 16)] + wbase
            be = _bucket(we, te)
            bo = _bucket(wo, to)
            idx_v[pl.ds(g * 16, 16)] = be * 9 + bo
        pltpu.async_copy(ptab_hbm.at[idx_v], rows_v, sem).wait()
        pltpu.sync_copy(rows_v, out_hbm.at[pl.ds(pbase + p0, _CH)])
        return carry

    lax.fori_loop(0, _NCH, chunk, 0)


_sc_lookup = pl.kernel(
    _body,
    out_type=jax.ShapeDtypeStruct((_B // 2, 2 * _EMB), jnp.float32),
    mesh=_mesh,
    scratch_types=[
        pltpu.VMEM((_PPW,), jnp.int32),
        pltpu.VMEM((_PPW,), jnp.int32),
        pltpu.VMEM((_PPW,), jnp.int32),
        pltpu.VMEM((_PPW,), jnp.int32),
        pltpu.VMEM((_CH,), jnp.int32),
        pltpu.VMEM((_CH, 2 * _EMB), jnp.float32),
        pltpu.SemaphoreType.DMA,
    ],
)


@jax.jit
def kernel(top_indices, distance_emb):
    emb = distance_emb.astype(jnp.float32)
    ptab = jnp.concatenate(
        [
            jnp.broadcast_to(emb[:, None, :], (9, 9, _EMB)),
            jnp.broadcast_to(emb[None, :, :], (9, 9, _EMB)),
        ],
        axis=-1,
    ).reshape(81, 2 * _EMB)
    top_flat = top_indices.reshape(-1).astype(jnp.int32)
    tope = top_flat[0::2]
    topo = top_flat[1::2]
    out = _sc_lookup(tope, topo, jnp.asarray(_WE), jnp.asarray(_WO), ptab)
    return out.reshape(_NWORDS, _K, _EMB)


# speedup vs baseline: 15.3653x; 6.8067x over previous
import functools
import numpy as np
import jax
import jax.numpy as jnp
from jax import lax
from jax.experimental import pallas as pl
from jax.experimental.pallas import tpu as pltpu
from jax.experimental.pallas import tpu_sc as plsc

_NWORDS = 8192
_K = 50
_EMB = 64
_NC, _NS = 2, 16
_NWK = _NC * _NS
_B = _NWORDS * _K
_BPW = _B // _NWK
_PPW = _BPW // 2
_CH = 128
_NCH = _PPW // _CH
_WPW = _BPW // _K

_WE = (2 * np.arange(_PPW, dtype=np.int32)) // _K
_WO = (2 * np.arange(_PPW, dtype=np.int32) + 1) // _K

_mesh = plsc.VectorSubcoreMesh(
    core_axis_name="c", subcore_axis_name="s", num_cores=_NC, num_subcores=_NS
)


def _bucket(word, top):
    d = jnp.maximum(word - top, 1)
    one = jnp.int32(1)
    zero = jnp.int32(0)
    bl = (
        4
        + jnp.where(d >= 8, one, zero)
        + jnp.where(d >= 16, one, zero)
        + jnp.where(d >= 32, one, zero)
        + jnp.where(d >= 64, one, zero)
    )
    return jnp.where(d < 5, d - 1, bl)


def _body(tope_hbm, topo_hbm, we_hbm, wo_hbm, ptab_hbm, out_hbm,
          tope_v, topo_v, we_v, wo_v, ptab_s, idx_v, rows_v, sem):
    sid = lax.axis_index("s")
    wid = sid * _NC + lax.axis_index("c")
    pbase = wid * _PPW
    pltpu.sync_copy(tope_hbm.at[pl.ds(pbase, _PPW)], tope_v)
    pltpu.sync_copy(topo_hbm.at[pl.ds(pbase, _PPW)], topo_v)
    pltpu.sync_copy(we_hbm, we_v)
    pltpu.sync_copy(wo_hbm, wo_v)

    @pl.when(sid == 0)
    def _():
        pltpu.sync_copy(ptab_hbm, ptab_s)

    plsc.subcore_barrier()
    wbase = wid * _WPW

    def chunk(c, carry):
        p0 = c * _CH
        for g in range(_CH // 16):
            j = p0 + g * 16
            te = tope_v[pl.ds(j, 16)]
            to = topo_v[pl.ds(j, 16)]
            we = we_v[pl.ds(j, 16)] + wbase
            wo = wo_v[pl.ds(j, 16)] + wbase
            be = _bucket(we, te)
            bo = _bucket(wo, to)
            idx_v[pl.ds(g * 16, 16)] = be * 9 + bo
        pltpu.async_copy(ptab_s.at[idx_v], rows_v, sem).wait()
        pltpu.sync_copy(rows_v, out_hbm.at[pl.ds(pbase + p0, _CH)])
        return carry

    lax.fori_loop(0, _NCH, chunk, 0)


_sc_lookup = pl.kernel(
    _body,
    out_type=jax.ShapeDtypeStruct((_B // 2, 2 * _EMB), jnp.float32),
    mesh=_mesh,
    scratch_types=[
        pltpu.VMEM((_PPW,), jnp.int32),
        pltpu.VMEM((_PPW,), jnp.int32),
        pltpu.VMEM((_PPW,), jnp.int32),
        pltpu.VMEM((_PPW,), jnp.int32),
        pltpu.VMEM_SHARED((81, 2 * _EMB), jnp.float32),
        pltpu.VMEM((_CH,), jnp.int32),
        pltpu.VMEM((_CH, 2 * _EMB), jnp.float32),
        pltpu.SemaphoreType.DMA,
    ],
)


@jax.jit
def kernel(top_indices, distance_emb):
    emb = distance_emb.astype(jnp.float32)
    ptab = jnp.concatenate(
        [
            jnp.broadcast_to(emb[:, None, :], (9, 9, _EMB)),
            jnp.broadcast_to(emb[None, :, :], (9, 9, _EMB)),
        ],
        axis=-1,
    ).reshape(81, 2 * _EMB)
    top_flat = top_indices.reshape(-1).astype(jnp.int32)
    tope = top_flat[0::2]
    topo = top_flat[1::2]
    out = _sc_lookup(tope, topo, jnp.asarray(_WE), jnp.asarray(_WO), ptab)
    return out.reshape(_NWORDS, _K, _EMB)


# speedup vs baseline: 15.7851x; 1.0273x over previous
import functools
import numpy as np
import jax
import jax.numpy as jnp
from jax import lax
from jax.experimental import pallas as pl
from jax.experimental.pallas import tpu as pltpu
from jax.experimental.pallas import tpu_sc as plsc

_NWORDS = 8192
_K = 50
_EMB = 64
_NC, _NS = 2, 16
_NWK = _NC * _NS
_B = _NWORDS * _K
_BPW = _B // _NWK
_PPW = _BPW // 2
_CH = 128
_NCH = _PPW // _CH
_WPW = _BPW // _K

_WE = (2 * np.arange(_PPW, dtype=np.int32)) // _K
_WO = (2 * np.arange(_PPW, dtype=np.int32) + 1) // _K

_mesh = plsc.VectorSubcoreMesh(
    core_axis_name="c", subcore_axis_name="s", num_cores=_NC, num_subcores=_NS
)


def _bucket(word, top):
    d = jnp.maximum(word - top, 1)
    one = jnp.int32(1)
    zero = jnp.int32(0)
    bl = (
        4
        + jnp.where(d >= 8, one, zero)
        + jnp.where(d >= 16, one, zero)
        + jnp.where(d >= 32, one, zero)
        + jnp.where(d >= 64, one, zero)
    )
    return jnp.where(d < 5, d - 1, bl)


def _body(tope_hbm, topo_hbm, we_hbm, wo_hbm, ptab_hbm, out_hbm,
          tope_v, topo_v, we_v, wo_v, ptab_s,
          idx0, idx1, rows0, rows1, gsem0, gsem1, osem0, osem1):
    sid = lax.axis_index("s")
    wid = sid * _NC + lax.axis_index("c")
    pbase = wid * _PPW
    pltpu.sync_copy(tope_hbm.at[pl.ds(pbase, _PPW)], tope_v)
    pltpu.sync_copy(topo_hbm.at[pl.ds(pbase, _PPW)], topo_v)
    pltpu.sync_copy(we_hbm, we_v)
    pltpu.sync_copy(wo_hbm, wo_v)

    @pl.when(sid == 0)
    def _():
        pltpu.sync_copy(ptab_hbm, ptab_s)

    plsc.subcore_barrier()
    wbase = wid * _WPW

    def compute_idx(c, idxbuf):
        p0 = c * _CH
        for g in range(_CH // 16):
            j = p0 + g * 16
            te = tope_v[pl.ds(j, 16)]
            to = topo_v[pl.ds(j, 16)]
            we = we_v[pl.ds(j, 16)] + wbase
            wo = wo_v[pl.ds(j, 16)] + wbase
            be = _bucket(we, te)
            bo = _bucket(wo, to)
            idxbuf[pl.ds(g * 16, 16)] = be * 9 + bo

    def out_ref(c):
        return out_hbm.at[pl.ds(pbase + c * _CH, _CH)]

    def start_gather(idxbuf, rowsbuf, gsem):
        pltpu.async_copy(ptab_s.at[idxbuf], rowsbuf, gsem)

    def wait_gather(idxbuf, rowsbuf, gsem):
        pltpu.make_async_copy(ptab_s.at[idxbuf], rowsbuf, gsem).wait()

    compute_idx(0, idx0)
    start_gather(idx0, rows0, gsem0)

    def step(i, carry):
        c = 2 * i
        compute_idx(c + 1, idx1)

        @pl.when(i >= 1)
        def _():
            pltpu.make_async_copy(rows1, out_ref(c - 1), osem1).wait()

        start_gather(idx1, rows1, gsem1)
        wait_gather(idx0, rows0, gsem0)
        pltpu.async_copy(rows0, out_ref(c), osem0)

        @pl.when(i < _NCH // 2 - 1)
        def _():
            compute_idx(c + 2, idx0)
            pltpu.make_async_copy(rows0, out_ref(c), osem0).wait()
            start_gather(idx0, rows0, gsem0)

        wait_gather(idx1, rows1, gsem1)
        pltpu.async_copy(rows1, out_ref(c + 1), osem1)
        return carry

    lax.fori_loop(0, _NCH // 2, step, 0)
    pltpu.make_async_copy(rows0, out_ref(_NCH - 2), osem0).wait()
    pltpu.make_async_copy(rows1, out_ref(_NCH - 1), osem1).wait()


_sc_lookup = pl.kernel(
    _body,
    out_type=jax.ShapeDtypeStruct((_B // 2, 2 * _EMB), jnp.float32),
    mesh=_mesh,
    scratch_types=[
        pltpu.VMEM((_PPW,), jnp.int32),
        pltpu.VMEM((_PPW,), jnp.int32),
        pltpu.VMEM((_PPW,), jnp.int32),
        pltpu.VMEM((_PPW,), jnp.int32),
        pltpu.VMEM_SHARED((81, 2 * _EMB), jnp.float32),
        pltpu.VMEM((_CH,), jnp.int32),
        pltpu.VMEM((_CH,), jnp.int32),
        pltpu.VMEM((_CH, 2 * _EMB), jnp.float32),
        pltpu.VMEM((_CH, 2 * _EMB), jnp.float32),
        pltpu.SemaphoreType.DMA,
        pltpu.SemaphoreType.DMA,
        pltpu.SemaphoreType.DMA,
        pltpu.SemaphoreType.DMA,
    ],
)


@jax.jit
def kernel(top_indices, distance_emb):
    emb = distance_emb.astype(jnp.float32)
    ptab = jnp.concatenate(
        [
            jnp.broadcast_to(emb[:, None, :], (9, 9, _EMB)),
            jnp.broadcast_to(emb[None, :, :], (9, 9, _EMB)),
        ],
        axis=-1,
    ).reshape(81, 2 * _EMB)
    top_flat = top_indices.reshape(-1).astype(jnp.int32)
    tope = top_flat[0::2]
    topo = top_flat[1::2]
    out = _sc_lookup(tope, topo, jnp.asarray(_WE), jnp.asarray(_WO), ptab)
    return out.reshape(_NWORDS, _K, _EMB)


# speedup vs baseline: 17.5325x; 1.1107x over previous
import functools
import numpy as np
import jax
import jax.numpy as jnp
from jax import lax
from jax.experimental import pallas as pl
from jax.experimental.pallas import tpu as pltpu
from jax.experimental.pallas import tpu_sc as plsc

_NWORDS = 8192
_K = 50
_EMB = 64
_NC, _NS = 2, 16
_NWK = _NC * _NS
_B = _NWORDS * _K
_BPW = _B // _NWK
_PPW = _BPW // 2
_CH = 128
_NCH = _PPW // _CH
_WPW = _BPW // _K

_WE = (2 * np.arange(_PPW, dtype=np.int32)) // _K
_WO = (2 * np.arange(_PPW, dtype=np.int32) + 1) // _K

_mesh = plsc.VectorSubcoreMesh(
    core_axis_name="c", subcore_axis_name="s", num_cores=_NC, num_subcores=_NS
)


def _bucket(word, top):
    d = jnp.maximum(word - top, 1)
    one = jnp.int32(1)
    zero = jnp.int32(0)
    bl = (
        4
        + jnp.where(d >= 8, one, zero)
        + jnp.where(d >= 16, one, zero)
        + jnp.where(d >= 32, one, zero)
        + jnp.where(d >= 64, one, zero)
    )
    return jnp.where(d < 5, d - 1, bl)


def _body(tope_hbm, topo_hbm, we_hbm, wo_hbm, ptab_hbm, out_hbm,
          tope_v, topo_v, we_v, wo_v, ptab_s,
          idx0, idx1, rows0, rows1, gsem0, gsem1, osem0, osem1):
    sid = lax.axis_index("s")
    wid = sid * _NC + lax.axis_index("c")
    pbase = wid * _PPW
    pltpu.sync_copy(tope_hbm.at[pl.ds(pbase, _PPW)], tope_v)
    pltpu.sync_copy(topo_hbm.at[pl.ds(pbase, _PPW)], topo_v)
    pltpu.sync_copy(we_hbm, we_v)
    pltpu.sync_copy(wo_hbm, wo_v)

    pltpu.sync_copy(ptab_hbm, ptab_s.at[pl.ds(sid * 81, 81)])
    sbase = sid * 81
    wbase = wid * _WPW

    def compute_idx(c, idxbuf):
        p0 = c * _CH
        for g in range(_CH // 16):
            j = p0 + g * 16
            te = tope_v[pl.ds(j, 16)]
            to = topo_v[pl.ds(j, 16)]
            we = we_v[pl.ds(j, 16)] + wbase
            wo = wo_v[pl.ds(j, 16)] + wbase
            be = _bucket(we, te)
            bo = _bucket(wo, to)
            idxbuf[pl.ds(g * 16, 16)] = be * 9 + bo + sbase

    def out_ref(c):
        return out_hbm.at[pl.ds(pbase + c * _CH, _CH)]

    def start_gather(idxbuf, rowsbuf, gsem):
        pltpu.async_copy(ptab_s.at[idxbuf], rowsbuf, gsem)

    def wait_gather(idxbuf, rowsbuf, gsem):
        pltpu.make_async_copy(ptab_s.at[idxbuf], rowsbuf, gsem).wait()

    compute_idx(0, idx0)
    start_gather(idx0, rows0, gsem0)

    def step(i, carry):
        c = 2 * i
        compute_idx(c + 1, idx1)

        @pl.when(i >= 1)
        def _():
            pltpu.make_async_copy(rows1, out_ref(c - 1), osem1).wait()

        start_gather(idx1, rows1, gsem1)
        wait_gather(idx0, rows0, gsem0)
        pltpu.async_copy(rows0, out_ref(c), osem0)

        @pl.when(i < _NCH // 2 - 1)
        def _():
            compute_idx(c + 2, idx0)
            pltpu.make_async_copy(rows0, out_ref(c), osem0).wait()
            start_gather(idx0, rows0, gsem0)

        wait_gather(idx1, rows1, gsem1)
        pltpu.async_copy(rows1, out_ref(c + 1), osem1)
        return carry

    lax.fori_loop(0, _NCH // 2, step, 0)
    pltpu.make_async_copy(rows0, out_ref(_NCH - 2), osem0).wait()
    pltpu.make_async_copy(rows1, out_ref(_NCH - 1), osem1).wait()


_sc_lookup = pl.kernel(
    _body,
    out_type=jax.ShapeDtypeStruct((_B // 2, 2 * _EMB), jnp.float32),
    mesh=_mesh,
    scratch_types=[
        pltpu.VMEM((_PPW,), jnp.int32),
        pltpu.VMEM((_PPW,), jnp.int32),
        pltpu.VMEM((_PPW,), jnp.int32),
        pltpu.VMEM((_PPW,), jnp.int32),
        pltpu.VMEM_SHARED((_NS * 81, 2 * _EMB), jnp.float32),
        pltpu.VMEM((_CH,), jnp.int32),
        pltpu.VMEM((_CH,), jnp.int32),
        pltpu.VMEM((_CH, 2 * _EMB), jnp.float32),
        pltpu.VMEM((_CH, 2 * _EMB), jnp.float32),
        pltpu.SemaphoreType.DMA,
        pltpu.SemaphoreType.DMA,
        pltpu.SemaphoreType.DMA,
        pltpu.SemaphoreType.DMA,
    ],
)


@jax.jit
def kernel(top_indices, distance_emb):
    emb = distance_emb.astype(jnp.float32)
    ptab = jnp.concatenate(
        [
            jnp.broadcast_to(emb[:, None, :], (9, 9, _EMB)),
            jnp.broadcast_to(emb[None, :, :], (9, 9, _EMB)),
        ],
        axis=-1,
    ).reshape(81, 2 * _EMB)
    top_flat = top_indices.reshape(-1).astype(jnp.int32)
    tope = top_flat[0::2]
    topo = top_flat[1::2]
    out = _sc_lookup(tope, topo, jnp.asarray(_WE), jnp.asarray(_WO), ptab)
    return out.reshape(_NWORDS, _K, _EMB)
